# baseline (speedup 1.0000x reference)
import functools

import jax
import jax.numpy as jnp
from jax import lax
from jax.experimental import pallas as pl
from jax.experimental.pallas import tpu as pltpu
from jax.experimental.pallas import tpu_sc as plsc

NSAMP = 64
NSTEP = 4
NPER = 16
BASEVAR = 64.0
NRAYS = 65536
L = 16
NC, NS = 2, 16
NWORK = NC * NS
GROUPS = NRAYS // L
GPW = GROUPS // NWORK
RPW = GPW * L
CH = 8
NCHUNK = GPW // CH
SOUT = NSAMP + NSTEP * NPER
NOUT = SOUT + 1


def _sqrtv(a):
    i = lax.bitcast_convert_type(a, jnp.int32)
    x = lax.bitcast_convert_type(jnp.int32(0x5F3759DF) - (i >> 1), jnp.float32)
    for _ in range(3):
        x = x * (1.5 - 0.5 * a * x * x)
    return a * x


def _search_right(ref, v, length, lane):
    pos = jnp.zeros((L,), jnp.int32)
    step = 64
    while step >= 1:
        cand = jnp.minimum(pos + step, length - 1)
        av = plsc.load_gather(ref, [cand * L + lane])
        pos = jnp.where(av <= v, cand, pos)
        step //= 2
    return pos + 1


def _sc_body(org_hbm, dir_hbm, near_hbm, far_hbm, out_hbm,
             org_v, dir_v, near_v, far_v, outc_v,
             spb_a, sdf_a, spb_b, sdf_b, cdf, delta):
    cid = lax.axis_index("c")
    sid = lax.axis_index("s")
    wid = sid * NC + cid
    lane = lax.iota(jnp.int32, L)
    zf = jnp.zeros((L,), jnp.float32)
    zi = jnp.zeros((L,), jnp.int32)
    onei = jnp.ones((L,), jnp.int32)

    pltpu.sync_copy(org_hbm.at[pl.ds(wid * (RPW * 3), RPW * 3)], org_v)
    pltpu.sync_copy(dir_hbm.at[pl.ds(wid * (RPW * 3), RPW * 3)], dir_v)
    pltpu.sync_copy(near_hbm.at[pl.ds(wid * RPW, RPW)], near_v)
    pltpu.sync_copy(far_hbm.at[pl.ds(wid * RPW, RPW)], far_v)

    def row(ref, i):
        return ref[pl.ds(i * L, L)]

    def setrow(ref, i, v):
        ref[pl.ds(i * L, L)] = v

    @plsc.parallel_loop(0, NOUT)
    def _(i):
        setrow(delta, i, zi)

    def do_group(g, gi):
        r3 = (g * L + lane) * 3
        ox = plsc.load_gather(org_v, [r3])
        oy = plsc.load_gather(org_v, [r3 + 1])
        oz = plsc.load_gather(org_v, [r3 + 2])
        dx = plsc.load_gather(dir_v, [r3])
        dy = plsc.load_gather(dir_v, [r3 + 1])
        dz = plsc.load_gather(dir_v, [r3 + 2])
        near = row(near_v, g)
        far = row(far_v, g)
        fmn = far - near

        def sdf_at(x):
            t = near + x * fmn
            px = ox + dx * t
            py = oy + dy * t
            pz = oz + dz * t
            return _sqrtv(px * px + py * py + pz * pz) - 1.0

        @plsc.parallel_loop(0, NSAMP)
        def _(j):
            xv = jnp.broadcast_to(
                lax.convert_element_type(j, jnp.float32) * (1.0 / NSAMP), (L,))
            setrow(spb_a, j, xv)
            setrow(sdf_a, j, sdf_at(xv))

        setrow(spb_a, NSAMP, jnp.full((L,), 1.0, jnp.float32))
        setrow(cdf, 0, zf)
        oray = gi * (L * NOUT) + lane * NOUT

        bufs = [(spb_a, sdf_a), (spb_b, sdf_b)]
        for it in range(NSTEP):
            S = NSAMP + NPER * it
            spb_src, sdf_src = bufs[it % 2]
            spb_dst, sdf_dst = bufs[(it + 1) % 2]
            last = it == NSTEP - 1
            inv_s = BASEVAR * (2.0 ** it)

            def passA(s, carry):
                trans, pcraw, c = carry
                sdf_s = row(sdf_src, s)
                spb_s = row(spb_src, s)
                sdf_n = row(sdf_src, s + 1)
                spb_n = row(spb_src, s + 1)
                dlt = (spb_n - spb_s) * fmn
                cos = (sdf_n - sdf_s) / (dlt + 1e-5)
                cv = jnp.clip(jnp.minimum(pcraw, cos), -1e3, 0.0)
                mid = (sdf_s + sdf_n) * 0.5
                h = cv * dlt * 0.5
                ea = jnp.exp(jnp.minimum((h - mid) * inv_s, 40.0))
                eb = jnp.exp(jnp.minimum((-h - mid) * inv_s, 40.0))
                q = 1e-5 * ((1.0 + ea) * (1.0 + eb))
                alpha = (eb - ea + q) / (1.0 + eb + q)
                c = c + alpha * trans + 1e-5
                trans = trans * (1.0 - alpha + 1e-7)
                setrow(cdf, s + 1, c)
                return (trans, cos, c)

            carry0 = (jnp.ones((L,), jnp.float32), zf, zf)
            _, _, c_end = plsc.parallel_loop(0, S - 1, carry=carry0)(passA)
            ws = c_end + 1e-5
            setrow(cdf, S, ws)

            def sample_one(kf):
                u = jnp.broadcast_to(kf * (1.0 / 17.0) + (1.0 / 34.0), (L,)) * ws
                inds = _search_right(cdf, u, S + 1, lane)
                below = (inds - 1) * L + lane
                above = inds * L + lane
                g0 = plsc.load_gather(cdf, [below])
                g1 = plsc.load_gather(cdf, [above])
                b0 = plsc.load_gather(spb_src, [below])
                b1 = plsc.load_gather(spb_src, [above])
                t = jnp.clip((u - g0) / (g1 - g0), 0.0, 1.0)
                return b0 + t * (b1 - b0), inds

            @plsc.parallel_loop(0, NPER, unroll=2)
            def _(k):
                bv, pb = sample_one(lax.convert_element_type(k, jnp.float32))
                plsc.addupdate_scatter(delta, [pb * L + lane], onei)
                rb = pb + jnp.broadcast_to(k, (L,)).astype(jnp.int32)
                if last:
                    plsc.store_scatter(outc_v, [oray + rb], near + bv * fmn)
                else:
                    plsc.store_scatter(spb_dst, [rb * L + lane], bv)
                    plsc.store_scatter(sdf_dst, [rb * L + lane], sdf_at(bv))
            end_b, _ = sample_one(jnp.float32(NPER))

            def apass(i, cnt):
                dv = row(delta, i)
                setrow(delta, i, zi)
                cnt = cnt + dv
                r = cnt + jnp.broadcast_to(i, (L,)).astype(jnp.int32)
                av = row(spb_src, i)
                if last:
                    plsc.store_scatter(outc_v, [oray + r], near + av * fmn)
                else:
                    plsc.store_scatter(spb_dst, [r * L + lane], av)
                    plsc.store_scatter(sdf_dst, [r * L + lane], row(sdf_src, i))
                return cnt

            plsc.parallel_loop(0, S, carry=zi)(apass)
            setrow(delta, S, zi)

            endv = jnp.maximum(row(spb_src, S), end_b)
            if last:
                plsc.store_scatter(
                    outc_v, [oray + jnp.broadcast_to(SOUT, (L,)).astype(jnp.int32)],
                    near + endv * fmn)
            else:
                setrow(spb_dst, S + NPER, endv)

    def chunk_body(c, _):
        def group_body(gg, _):
            do_group(c * CH + gg, gg)
            return 0

        lax.fori_loop(0, CH, group_body, 0)
        pltpu.sync_copy(
            outc_v,
            out_hbm.at[pl.ds((wid * GPW + c * CH) * (L * NOUT), CH * L * NOUT)])
        return 0

    lax.fori_loop(0, NCHUNK, chunk_body, 0)


_mesh = plsc.VectorSubcoreMesh(core_axis_name="c", subcore_axis_name="s")

_sc_sampler = functools.partial(
    pl.kernel,
    out_type=jax.ShapeDtypeStruct((NRAYS * NOUT,), jnp.float32),
    mesh=_mesh,
    compiler_params=pltpu.CompilerParams(needs_layout_passes=False),
    scratch_types=[
        pltpu.VMEM((RPW * 3,), jnp.float32),
        pltpu.VMEM((RPW * 3,), jnp.float32),
        pltpu.VMEM((RPW,), jnp.float32),
        pltpu.VMEM((RPW,), jnp.float32),
        pltpu.VMEM((CH * L * NOUT,), jnp.float32),
        pltpu.VMEM((NOUT * L,), jnp.float32),
        pltpu.VMEM((SOUT * L,), jnp.float32),
        pltpu.VMEM((NOUT * L,), jnp.float32),
        pltpu.VMEM((SOUT * L,), jnp.float32),
        pltpu.VMEM((NOUT * L,), jnp.float32),
        pltpu.VMEM((NOUT * L,), jnp.int32),
    ],
)(_sc_body)


def kernel(origins, directions, nears, fars):
    out = _sc_sampler(origins.reshape(-1), directions.reshape(-1),
                      nears.reshape(-1), fars.reshape(-1))
    return out.reshape(NRAYS, NOUT)


# speedup vs baseline: 1.0565x; 1.0565x over previous
import functools

import jax
import jax.numpy as jnp
from jax import lax
from jax.experimental import pallas as pl
from jax.experimental.pallas import tpu as pltpu
from jax.experimental.pallas import tpu_sc as plsc

NSAMP = 64
NSTEP = 4
NPER = 16
BASEVAR = 64.0
NRAYS = 65536
L = 16
NC, NS = 2, 16
NWORK = NC * NS
GROUPS = NRAYS // L
GPW = GROUPS // NWORK
RPW = GPW * L
CH = 8
NCHUNK = GPW // CH
SOUT = NSAMP + NSTEP * NPER
NOUT = SOUT + 1


def _sqrtv(a):
    i = lax.bitcast_convert_type(a, jnp.int32)
    x = lax.bitcast_convert_type(jnp.int32(0x5F3759DF) - (i >> 1), jnp.float32)
    for _ in range(3):
        x = x * (1.5 - 0.5 * a * x * x)
    return a * x


def _search_right(ref, v, length, lane):
    pos = jnp.zeros((L,), jnp.int32)
    step = 64
    while step >= 1:
        cand = jnp.minimum(pos + step, length - 1)
        av = plsc.load_gather(ref, [cand * L + lane])
        pos = jnp.where(av <= v, cand, pos)
        step //= 2
    return pos + 1


def _sc_body(org_hbm, dir_hbm, near_hbm, far_hbm, out_hbm,
             org_v, dir_v, near_v, far_v, outc_v,
             spb_a, sdf_a, spb_b, sdf_b, cdf, delta):
    cid = lax.axis_index("c")
    sid = lax.axis_index("s")
    wid = sid * NC + cid
    lane = lax.iota(jnp.int32, L)
    zf = jnp.zeros((L,), jnp.float32)
    zi = jnp.zeros((L,), jnp.int32)
    onei = jnp.ones((L,), jnp.int32)

    pltpu.sync_copy(org_hbm.at[pl.ds(wid * (RPW * 3), RPW * 3)], org_v)
    pltpu.sync_copy(dir_hbm.at[pl.ds(wid * (RPW * 3), RPW * 3)], dir_v)
    pltpu.sync_copy(near_hbm.at[pl.ds(wid * RPW, RPW)], near_v)
    pltpu.sync_copy(far_hbm.at[pl.ds(wid * RPW, RPW)], far_v)

    def row(ref, i):
        return ref[pl.ds(i * L, L)]

    def setrow(ref, i, v):
        ref[pl.ds(i * L, L)] = v

    @plsc.parallel_loop(0, NOUT)
    def _(i):
        setrow(delta, i, zi)

    def do_group(g, gi):
        r3 = (g * L + lane) * 3
        ox = plsc.load_gather(org_v, [r3])
        oy = plsc.load_gather(org_v, [r3 + 1])
        oz = plsc.load_gather(org_v, [r3 + 2])
        dx = plsc.load_gather(dir_v, [r3])
        dy = plsc.load_gather(dir_v, [r3 + 1])
        dz = plsc.load_gather(dir_v, [r3 + 2])
        near = row(near_v, g)
        far = row(far_v, g)
        fmn = far - near

        def sdf_at(x):
            t = near + x * fmn
            px = ox + dx * t
            py = oy + dy * t
            pz = oz + dz * t
            return _sqrtv(px * px + py * py + pz * pz) - 1.0

        @plsc.parallel_loop(0, NSAMP)
        def _(j):
            xv = jnp.broadcast_to(
                lax.convert_element_type(j, jnp.float32) * (1.0 / NSAMP), (L,))
            setrow(spb_a, j, xv)
            setrow(sdf_a, j, sdf_at(xv))

        setrow(spb_a, NSAMP, jnp.full((L,), 1.0, jnp.float32))
        setrow(cdf, 0, zf)
        oray = gi * (L * NOUT) + lane * NOUT

        bufs = [(spb_a, sdf_a), (spb_b, sdf_b)]
        for it in range(NSTEP):
            S = NSAMP + NPER * it
            spb_src, sdf_src = bufs[it % 2]
            spb_dst, sdf_dst = bufs[(it + 1) % 2]
            last = it == NSTEP - 1
            inv_s = BASEVAR * (2.0 ** it)

            def passA(s, carry):
                trans, pcraw, c, sdf_s, spb_s = carry
                sdf_n = row(sdf_src, s + 1)
                spb_n = row(spb_src, s + 1)
                dlt = (spb_n - spb_s) * fmn
                cos = (sdf_n - sdf_s) / (dlt + 1e-5)
                cv = jnp.clip(jnp.minimum(pcraw, cos), -1e3, 0.0)
                mid = (sdf_s + sdf_n) * 0.5
                h = cv * dlt * 0.5
                ea = jnp.exp(jnp.minimum((h - mid) * inv_s, 40.0))
                eb = jnp.exp(jnp.minimum((-h - mid) * inv_s, 40.0))
                q = 1e-5 * ((1.0 + ea) * (1.0 + eb))
                alpha = (eb - ea + q) / (1.0 + eb + q)
                c = c + alpha * trans + 1e-5
                trans = trans * (1.0 - alpha + 1e-7)
                setrow(cdf, s + 1, c)
                return (trans, cos, c, sdf_n, spb_n)

            carry0 = (jnp.ones((L,), jnp.float32), zf, zf,
                      row(sdf_src, 0), row(spb_src, 0))
            _, _, c_end, _, _ = plsc.parallel_loop(0, S - 1, carry=carry0)(passA)
            ws = c_end + 1e-5
            setrow(cdf, S, ws)

            def sample_one(kf):
                u = jnp.broadcast_to(kf * (1.0 / 17.0) + (1.0 / 34.0), (L,)) * ws
                inds = _search_right(cdf, u, S + 1, lane)
                below = (inds - 1) * L + lane
                above = inds * L + lane
                g0 = plsc.load_gather(cdf, [below])
                g1 = plsc.load_gather(cdf, [above])
                b0 = plsc.load_gather(spb_src, [below])
                b1 = plsc.load_gather(spb_src, [above])
                t = jnp.clip((u - g0) / (g1 - g0), 0.0, 1.0)
                return b0 + t * (b1 - b0), inds

            @plsc.parallel_loop(0, NPER)
            def _(k):
                bv, pb = sample_one(lax.convert_element_type(k, jnp.float32))
                plsc.addupdate_scatter(delta, [pb * L + lane], onei)
                rb = pb + jnp.broadcast_to(k, (L,)).astype(jnp.int32)
                if last:
                    plsc.store_scatter(outc_v, [oray + rb], near + bv * fmn)
                else:
                    plsc.store_scatter(spb_dst, [rb * L + lane], bv)
                    plsc.store_scatter(sdf_dst, [rb * L + lane], sdf_at(bv))
            end_b, _ = sample_one(jnp.float32(NPER))

            def apass(i, cnt):
                dv = row(delta, i)
                setrow(delta, i, zi)
                cnt = cnt + dv
                r = cnt + jnp.broadcast_to(i, (L,)).astype(jnp.int32)
                av = row(spb_src, i)
                if last:
                    plsc.store_scatter(outc_v, [oray + r], near + av * fmn)
                else:
                    plsc.store_scatter(spb_dst, [r * L + lane], av)
                    plsc.store_scatter(sdf_dst, [r * L + lane], row(sdf_src, i))
                return cnt

            plsc.parallel_loop(0, S, carry=zi)(apass)
            setrow(delta, S, zi)

            endv = jnp.maximum(row(spb_src, S), end_b)
            if last:
                plsc.store_scatter(
                    outc_v, [oray + jnp.broadcast_to(SOUT, (L,)).astype(jnp.int32)],
                    near + endv * fmn)
            else:
                setrow(spb_dst, S + NPER, endv)

    def chunk_body(c, _):
        def group_body(gg, _):
            do_group(c * CH + gg, gg)
            return 0

        lax.fori_loop(0, CH, group_body, 0)
        pltpu.sync_copy(
            outc_v,
            out_hbm.at[pl.ds((wid * GPW + c * CH) * (L * NOUT), CH * L * NOUT)])
        return 0

    lax.fori_loop(0, NCHUNK, chunk_body, 0)


_mesh = plsc.VectorSubcoreMesh(core_axis_name="c", subcore_axis_name="s")

_sc_sampler = functools.partial(
    pl.kernel,
    out_type=jax.ShapeDtypeStruct((NRAYS * NOUT,), jnp.float32),
    mesh=_mesh,
    compiler_params=pltpu.CompilerParams(needs_layout_passes=False),
    scratch_types=[
        pltpu.VMEM((RPW * 3,), jnp.float32),
        pltpu.VMEM((RPW * 3,), jnp.float32),
        pltpu.VMEM((RPW,), jnp.float32),
        pltpu.VMEM((RPW,), jnp.float32),
        pltpu.VMEM((CH * L * NOUT,), jnp.float32),
        pltpu.VMEM((NOUT * L,), jnp.float32),
        pltpu.VMEM((SOUT * L,), jnp.float32),
        pltpu.VMEM((NOUT * L,), jnp.float32),
        pltpu.VMEM((SOUT * L,), jnp.float32),
        pltpu.VMEM((NOUT * L,), jnp.float32),
        pltpu.VMEM((NOUT * L,), jnp.int32),
    ],
)(_sc_body)


def kernel(origins, directions, nears, fars):
    out = _sc_sampler(origins.reshape(-1), directions.reshape(-1),
                      nears.reshape(-1), fars.reshape(-1))
    return out.reshape(NRAYS, NOUT)


# speedup vs baseline: 1.0819x; 1.0241x over previous
import functools

import jax
import jax.numpy as jnp
from jax import lax
from jax.experimental import pallas as pl
from jax.experimental.pallas import tpu as pltpu
from jax.experimental.pallas import tpu_sc as plsc

NSAMP = 64
NSTEP = 4
NPER = 16
BASEVAR = 64.0
NRAYS = 65536
L = 16
NC, NS = 2, 16
NWORK = NC * NS
GROUPS = NRAYS // L
GPW = GROUPS // NWORK
RPW = GPW * L
CH = 8
NCHUNK = GPW // CH
SOUT = NSAMP + NSTEP * NPER
NOUT = SOUT + 1


def _sqrtv(a):
    i = lax.bitcast_convert_type(a, jnp.int32)
    x = lax.bitcast_convert_type(jnp.int32(0x5F3759DF) - (i >> 1), jnp.float32)
    for _ in range(3):
        x = x * (1.5 - 0.5 * a * x * x)
    return a * x


def _search_right(ref, v, length, lane):
    pos = jnp.zeros((L,), jnp.int32)
    step = 64
    while step >= 1:
        cand = jnp.minimum(pos + step, length - 1)
        av = plsc.load_gather(ref, [cand * L + lane])
        pos = jnp.where(av <= v, cand, pos)
        step //= 2
    return pos + 1


def _sc_body(org_hbm, dir_hbm, near_hbm, far_hbm, out_hbm,
             org_v, dir_v, near_v, far_v, outc_v,
             spb_a, sdf_a, spb_b, sdf_b, cdf, delta):
    cid = lax.axis_index("c")
    sid = lax.axis_index("s")
    wid = sid * NC + cid
    lane = lax.iota(jnp.int32, L)
    zf = jnp.zeros((L,), jnp.float32)
    zi = jnp.zeros((L,), jnp.int32)
    onei = jnp.ones((L,), jnp.int32)

    pltpu.sync_copy(org_hbm.at[pl.ds(wid * (RPW * 3), RPW * 3)], org_v)
    pltpu.sync_copy(dir_hbm.at[pl.ds(wid * (RPW * 3), RPW * 3)], dir_v)
    pltpu.sync_copy(near_hbm.at[pl.ds(wid * RPW, RPW)], near_v)
    pltpu.sync_copy(far_hbm.at[pl.ds(wid * RPW, RPW)], far_v)

    def row(ref, i):
        return ref[pl.ds(i * L, L)]

    def setrow(ref, i, v):
        ref[pl.ds(i * L, L)] = v

    @plsc.parallel_loop(0, NOUT)
    def _(i):
        setrow(delta, i, zi)

    def do_group(g, gi):
        r3 = (g * L + lane) * 3
        ox = plsc.load_gather(org_v, [r3])
        oy = plsc.load_gather(org_v, [r3 + 1])
        oz = plsc.load_gather(org_v, [r3 + 2])
        dx = plsc.load_gather(dir_v, [r3])
        dy = plsc.load_gather(dir_v, [r3 + 1])
        dz = plsc.load_gather(dir_v, [r3 + 2])
        near = row(near_v, g)
        far = row(far_v, g)
        fmn = far - near

        def sdf_at(x):
            t = near + x * fmn
            px = ox + dx * t
            py = oy + dy * t
            pz = oz + dz * t
            return _sqrtv(px * px + py * py + pz * pz) - 1.0

        @plsc.parallel_loop(0, NSAMP)
        def _(j):
            xv = jnp.broadcast_to(
                lax.convert_element_type(j, jnp.float32) * (1.0 / NSAMP), (L,))
            setrow(spb_a, j, xv)
            setrow(sdf_a, j, sdf_at(xv))

        setrow(spb_a, NSAMP, jnp.full((L,), 1.0, jnp.float32))
        setrow(cdf, 0, zf)
        oray = gi * (L * NOUT) + lane * NOUT

        bufs = [(spb_a, sdf_a), (spb_b, sdf_b)]
        for it in range(NSTEP):
            S = NSAMP + NPER * it
            spb_src, sdf_src = bufs[it % 2]
            spb_dst, sdf_dst = bufs[(it + 1) % 2]
            last = it == NSTEP - 1
            inv_s = BASEVAR * (2.0 ** it)

            def passA(s, carry):
                trans, pcraw, c = carry
                sdf_s = row(sdf_src, s)
                spb_s = row(spb_src, s)
                sdf_n = row(sdf_src, s + 1)
                spb_n = row(spb_src, s + 1)
                dlt = (spb_n - spb_s) * fmn
                cos = (sdf_n - sdf_s) / (dlt + 1e-5)
                cv = jnp.clip(jnp.minimum(pcraw, cos), -1e3, 0.0)
                mid = (sdf_s + sdf_n) * 0.5
                h = cv * dlt * 0.5
                ea = jnp.exp(jnp.minimum((h - mid) * inv_s, 40.0))
                eb = jnp.exp(jnp.minimum((-h - mid) * inv_s, 40.0))
                q = 1e-5 * ((1.0 + ea) * (1.0 + eb))
                alpha = (eb - ea + q) / (1.0 + eb + q)
                c = c + alpha * trans + 1e-5
                trans = trans * (1.0 - alpha + 1e-7)
                setrow(cdf, s + 1, c)
                return (trans, cos, c)

            carry0 = (jnp.ones((L,), jnp.float32), zf, zf)
            _, _, c_end = plsc.parallel_loop(0, S - 1, carry=carry0)(passA)
            ws = c_end + 1e-5
            setrow(cdf, S, ws)

            def sample_one(kf):
                u = jnp.broadcast_to(kf * (1.0 / 17.0) + (1.0 / 34.0), (L,)) * ws
                inds = _search_right(cdf, u, S + 1, lane)
                below = (inds - 1) * L + lane
                above = inds * L + lane
                g0 = plsc.load_gather(cdf, [below])
                g1 = plsc.load_gather(cdf, [above])
                b0 = plsc.load_gather(spb_src, [below])
                b1 = plsc.load_gather(spb_src, [above])
                t = jnp.clip((u - g0) / (g1 - g0), 0.0, 1.0)
                return b0 + t * (b1 - b0), inds

            @plsc.parallel_loop(0, NPER)
            def _(k):
                bv, pb = sample_one(lax.convert_element_type(k, jnp.float32))
                plsc.addupdate_scatter(delta, [pb * L + lane], onei)
                rb = pb + jnp.broadcast_to(k, (L,)).astype(jnp.int32)
                if last:
                    plsc.store_scatter(outc_v, [oray + rb], near + bv * fmn)
                else:
                    plsc.store_scatter(spb_dst, [rb * L + lane], bv)
                    plsc.store_scatter(sdf_dst, [rb * L + lane], sdf_at(bv))
            end_b, _ = sample_one(jnp.float32(NPER))

            def apass(i, cnt):
                dv = row(delta, i)
                setrow(delta, i, zi)
                cnt = cnt + dv
                r = cnt + jnp.broadcast_to(i, (L,)).astype(jnp.int32)
                av = row(spb_src, i)
                if last:
                    plsc.store_scatter(outc_v, [oray + r], near + av * fmn)
                else:
                    plsc.store_scatter(spb_dst, [r * L + lane], av)
                    plsc.store_scatter(sdf_dst, [r * L + lane], row(sdf_src, i))
                return cnt

            plsc.parallel_loop(0, S, carry=zi)(apass)
            setrow(delta, S, zi)

            endv = jnp.maximum(row(spb_src, S), end_b)
            if last:
                plsc.store_scatter(
                    outc_v, [oray + jnp.broadcast_to(SOUT, (L,)).astype(jnp.int32)],
                    near + endv * fmn)
            else:
                setrow(spb_dst, S + NPER, endv)

    def chunk_body(c, _):
        def group_body(gg, _):
            do_group(c * CH + gg, gg)
            return 0

        lax.fori_loop(0, CH, group_body, 0)
        pltpu.sync_copy(
            outc_v,
            out_hbm.at[pl.ds((wid * GPW + c * CH) * (L * NOUT), CH * L * NOUT)])
        return 0

    lax.fori_loop(0, NCHUNK, chunk_body, 0)


_mesh = plsc.VectorSubcoreMesh(core_axis_name="c", subcore_axis_name="s")

_sc_sampler = functools.partial(
    pl.kernel,
    out_type=jax.ShapeDtypeStruct((NRAYS * NOUT,), jnp.float32),
    mesh=_mesh,
    compiler_params=pltpu.CompilerParams(needs_layout_passes=False),
    scratch_types=[
        pltpu.VMEM((RPW * 3,), jnp.float32),
        pltpu.VMEM((RPW * 3,), jnp.float32),
        pltpu.VMEM((RPW,), jnp.float32),
        pltpu.VMEM((RPW,), jnp.float32),
        pltpu.VMEM((CH * L * NOUT,), jnp.float32),
        pltpu.VMEM((NOUT * L,), jnp.float32),
        pltpu.VMEM((SOUT * L,), jnp.float32),
        pltpu.VMEM((NOUT * L,), jnp.float32),
        pltpu.VMEM((SOUT * L,), jnp.float32),
        pltpu.VMEM((NOUT * L,), jnp.float32),
        pltpu.VMEM((NOUT * L,), jnp.int32),
    ],
)(_sc_body)


def kernel(origins, directions, nears, fars):
    out = _sc_sampler(origins.reshape(-1), directions.reshape(-1),
                      nears.reshape(-1), fars.reshape(-1))
    return out.reshape(NRAYS, NOUT)


# speedup vs baseline: 1.0996x; 1.0164x over previous
import functools

import jax
import jax.numpy as jnp
from jax import lax
from jax.experimental import pallas as pl
from jax.experimental.pallas import tpu as pltpu
from jax.experimental.pallas import tpu_sc as plsc

NSAMP = 64
NSTEP = 4
NPER = 16
BASEVAR = 64.0
NRAYS = 65536
L = 16
NC, NS = 2, 16
NWORK = NC * NS
GROUPS = NRAYS // L
GPW = GROUPS // NWORK
RPW = GPW * L
CH = 16
NCHUNK = GPW // CH
SOUT = NSAMP + NSTEP * NPER
NOUT = SOUT + 1


def _sqrtv(a):
    i = lax.bitcast_convert_type(a, jnp.int32)
    x = lax.bitcast_convert_type(jnp.int32(0x5F3759DF) - (i >> 1), jnp.float32)
    for _ in range(3):
        x = x * (1.5 - 0.5 * a * x * x)
    return a * x


def _search_right(ref, v, length, lane):
    pos = jnp.zeros((L,), jnp.int32)
    step = 64
    while step >= 1:
        cand = jnp.minimum(pos + step, length - 1)
        av = plsc.load_gather(ref, [cand * L + lane])
        pos = jnp.where(av <= v, cand, pos)
        step //= 2
    return pos + 1


def _sc_body(org_hbm, dir_hbm, near_hbm, far_hbm, out_hbm,
             org_v, dir_v, near_v, far_v, outc_v,
             spb_a, sdf_a, spb_b, sdf_b, cdf, delta):
    cid = lax.axis_index("c")
    sid = lax.axis_index("s")
    wid = sid * NC + cid
    lane = lax.iota(jnp.int32, L)
    zf = jnp.zeros((L,), jnp.float32)
    zi = jnp.zeros((L,), jnp.int32)
    onei = jnp.ones((L,), jnp.int32)

    pltpu.sync_copy(org_hbm.at[pl.ds(wid * (RPW * 3), RPW * 3)], org_v)
    pltpu.sync_copy(dir_hbm.at[pl.ds(wid * (RPW * 3), RPW * 3)], dir_v)
    pltpu.sync_copy(near_hbm.at[pl.ds(wid * RPW, RPW)], near_v)
    pltpu.sync_copy(far_hbm.at[pl.ds(wid * RPW, RPW)], far_v)

    def row(ref, i):
        return ref[pl.ds(i * L, L)]

    def setrow(ref, i, v):
        ref[pl.ds(i * L, L)] = v

    @plsc.parallel_loop(0, NOUT)
    def _(i):
        setrow(delta, i, zi)

    def do_group(g, gi):
        r3 = (g * L + lane) * 3
        ox = plsc.load_gather(org_v, [r3])
        oy = plsc.load_gather(org_v, [r3 + 1])
        oz = plsc.load_gather(org_v, [r3 + 2])
        dx = plsc.load_gather(dir_v, [r3])
        dy = plsc.load_gather(dir_v, [r3 + 1])
        dz = plsc.load_gather(dir_v, [r3 + 2])
        near = row(near_v, g)
        far = row(far_v, g)
        fmn = far - near

        def sdf_at(x):
            t = near + x * fmn
            px = ox + dx * t
            py = oy + dy * t
            pz = oz + dz * t
            return _sqrtv(px * px + py * py + pz * pz) - 1.0

        @plsc.parallel_loop(0, NSAMP)
        def _(j):
            xv = jnp.broadcast_to(
                lax.convert_element_type(j, jnp.float32) * (1.0 / NSAMP), (L,))
            setrow(spb_a, j, xv)
            setrow(sdf_a, j, sdf_at(xv))

        setrow(spb_a, NSAMP, jnp.full((L,), 1.0, jnp.float32))
        setrow(cdf, 0, zf)
        oray = gi * (L * NOUT) + lane * NOUT

        bufs = [(spb_a, sdf_a), (spb_b, sdf_b)]
        for it in range(NSTEP):
            S = NSAMP + NPER * it
            spb_src, sdf_src = bufs[it % 2]
            spb_dst, sdf_dst = bufs[(it + 1) % 2]
            last = it == NSTEP - 1
            inv_s = BASEVAR * (2.0 ** it)

            def passA(s, carry):
                trans, pcraw, c = carry
                sdf_s = row(sdf_src, s)
                spb_s = row(spb_src, s)
                sdf_n = row(sdf_src, s + 1)
                spb_n = row(spb_src, s + 1)
                dlt = (spb_n - spb_s) * fmn
                cos = (sdf_n - sdf_s) / (dlt + 1e-5)
                cv = jnp.clip(jnp.minimum(pcraw, cos), -1e3, 0.0)
                mid = (sdf_s + sdf_n) * 0.5
                h = cv * dlt * 0.5
                ea = jnp.exp(jnp.minimum((h - mid) * inv_s, 40.0))
                eb = jnp.exp(jnp.minimum((-h - mid) * inv_s, 40.0))
                q = 1e-5 * ((1.0 + ea) * (1.0 + eb))
                alpha = (eb - ea + q) / (1.0 + eb + q)
                c = c + alpha * trans + 1e-5
                trans = trans * ((1.0 + 1e-7) - alpha)
                setrow(cdf, s + 1, c)
                return (trans, cos, c)

            carry0 = (jnp.ones((L,), jnp.float32), zf, zf)
            _, _, c_end = plsc.parallel_loop(0, S - 1, carry=carry0)(passA)
            ws = c_end + 1e-5
            setrow(cdf, S, ws)

            def sample_one(kf):
                u = jnp.broadcast_to(kf * (1.0 / 17.0) + (1.0 / 34.0), (L,)) * ws
                inds = _search_right(cdf, u, S + 1, lane)
                below = (inds - 1) * L + lane
                above = inds * L + lane
                g0 = plsc.load_gather(cdf, [below])
                g1 = plsc.load_gather(cdf, [above])
                b0 = plsc.load_gather(spb_src, [below])
                b1 = plsc.load_gather(spb_src, [above])
                t = jnp.clip((u - g0) / (g1 - g0), 0.0, 1.0)
                return b0 + t * (b1 - b0), inds

            @plsc.parallel_loop(0, NPER)
            def _(k):
                bv, pb = sample_one(lax.convert_element_type(k, jnp.float32))
                plsc.addupdate_scatter(delta, [pb * L + lane], onei)
                rb = pb + jnp.broadcast_to(k, (L,)).astype(jnp.int32)
                if last:
                    plsc.store_scatter(outc_v, [oray + rb], near + bv * fmn)
                else:
                    plsc.store_scatter(spb_dst, [rb * L + lane], bv)
                    plsc.store_scatter(sdf_dst, [rb * L + lane], sdf_at(bv))
            end_b, _ = sample_one(jnp.float32(NPER))

            def apass(i, cnt):
                dv = row(delta, i)
                setrow(delta, i, zi)
                cnt = cnt + dv
                r = cnt + jnp.broadcast_to(i, (L,)).astype(jnp.int32)
                av = row(spb_src, i)
                if last:
                    plsc.store_scatter(outc_v, [oray + r], near + av * fmn)
                else:
                    plsc.store_scatter(spb_dst, [r * L + lane], av)
                    plsc.store_scatter(sdf_dst, [r * L + lane], row(sdf_src, i))
                return cnt

            plsc.parallel_loop(0, S, carry=zi)(apass)
            setrow(delta, S, zi)

            endv = jnp.maximum(row(spb_src, S), end_b)
            if last:
                plsc.store_scatter(
                    outc_v, [oray + jnp.broadcast_to(SOUT, (L,)).astype(jnp.int32)],
                    near + endv * fmn)
            else:
                setrow(spb_dst, S + NPER, endv)

    def chunk_body(c, _):
        def group_body(gg, _):
            do_group(c * CH + gg, gg)
            return 0

        lax.fori_loop(0, CH, group_body, 0)
        pltpu.sync_copy(
            outc_v,
            out_hbm.at[pl.ds((wid * GPW + c * CH) * (L * NOUT), CH * L * NOUT)])
        return 0

    lax.fori_loop(0, NCHUNK, chunk_body, 0)


_mesh = plsc.VectorSubcoreMesh(core_axis_name="c", subcore_axis_name="s")

_sc_sampler = functools.partial(
    pl.kernel,
    out_type=jax.ShapeDtypeStruct((NRAYS * NOUT,), jnp.float32),
    mesh=_mesh,
    compiler_params=pltpu.CompilerParams(needs_layout_passes=False),
    scratch_types=[
        pltpu.VMEM((RPW * 3,), jnp.float32),
        pltpu.VMEM((RPW * 3,), jnp.float32),
        pltpu.VMEM((RPW,), jnp.float32),
        pltpu.VMEM((RPW,), jnp.float32),
        pltpu.VMEM((CH * L * NOUT,), jnp.float32),
        pltpu.VMEM((NOUT * L,), jnp.float32),
        pltpu.VMEM((SOUT * L,), jnp.float32),
        pltpu.VMEM((NOUT * L,), jnp.float32),
        pltpu.VMEM((SOUT * L,), jnp.float32),
        pltpu.VMEM((NOUT * L,), jnp.float32),
        pltpu.VMEM((NOUT * L,), jnp.int32),
    ],
)(_sc_body)


def kernel(origins, directions, nears, fars):
    out = _sc_sampler(origins.reshape(-1), directions.reshape(-1),
                      nears.reshape(-1), fars.reshape(-1))
    return out.reshape(NRAYS, NOUT)


# speedup vs baseline: 1.1007x; 1.0010x over previous
import functools

import jax
import jax.numpy as jnp
from jax import lax
from jax.experimental import pallas as pl
from jax.experimental.pallas import tpu as pltpu
from jax.experimental.pallas import tpu_sc as plsc

NSAMP = 64
NSTEP = 4
NPER = 16
BASEVAR = 64.0
NRAYS = 65536
L = 16
NC, NS = 2, 16
NWORK = NC * NS
GROUPS = NRAYS // L
GPW = GROUPS // NWORK
RPW = GPW * L
CH = 16
NCHUNK = GPW // CH
SOUT = NSAMP + NSTEP * NPER
NOUT = SOUT + 1


def _sqrtv(a):
    i = lax.bitcast_convert_type(a, jnp.int32)
    x = lax.bitcast_convert_type(jnp.int32(0x5F3759DF) - (i >> 1), jnp.float32)
    for _ in range(3):
        x = x * (1.5 - 0.5 * a * x * x)
    return a * x


def _search_right(ref, v, length, lane):
    pos = jnp.zeros((L,), jnp.int32)
    step = 64
    while step >= 1:
        cand = jnp.minimum(pos + step, length - 1)
        av = plsc.load_gather(ref, [cand * L + lane])
        pos = jnp.where(av <= v, cand, pos)
        step //= 2
    return pos + 1


def _sc_body(org_hbm, dir_hbm, near_hbm, far_hbm, out_hbm,
             org_v, dir_v, near_v, far_v, outc_v,
             spb_a, sdf_a, spb_b, sdf_b, cdf, delta, ubin, sdf0):
    cid = lax.axis_index("c")
    sid = lax.axis_index("s")
    wid = sid * NC + cid
    lane = lax.iota(jnp.int32, L)
    zf = jnp.zeros((L,), jnp.float32)
    zi = jnp.zeros((L,), jnp.int32)
    onei = jnp.ones((L,), jnp.int32)

    pltpu.sync_copy(org_hbm.at[pl.ds(wid * (RPW * 3), RPW * 3)], org_v)
    pltpu.sync_copy(dir_hbm.at[pl.ds(wid * (RPW * 3), RPW * 3)], dir_v)
    pltpu.sync_copy(near_hbm.at[pl.ds(wid * RPW, RPW)], near_v)
    pltpu.sync_copy(far_hbm.at[pl.ds(wid * RPW, RPW)], far_v)

    def row(ref, i):
        return ref[pl.ds(i * L, L)]

    def setrow(ref, i, v):
        ref[pl.ds(i * L, L)] = v

    @plsc.parallel_loop(0, NOUT)
    def _(i):
        setrow(delta, i, zi)

    @plsc.parallel_loop(0, NSAMP + 1)
    def _(j):
        setrow(ubin, j, jnp.broadcast_to(
            lax.convert_element_type(j, jnp.float32) * (1.0 / NSAMP), (L,)))

    def do_group(g, gi):
        r3 = (g * L + lane) * 3
        ox = plsc.load_gather(org_v, [r3])
        oy = plsc.load_gather(org_v, [r3 + 1])
        oz = plsc.load_gather(org_v, [r3 + 2])
        dx = plsc.load_gather(dir_v, [r3])
        dy = plsc.load_gather(dir_v, [r3 + 1])
        dz = plsc.load_gather(dir_v, [r3 + 2])
        near = row(near_v, g)
        far = row(far_v, g)
        fmn = far - near

        def sdf_at(x):
            t = near + x * fmn
            px = ox + dx * t
            py = oy + dy * t
            pz = oz + dz * t
            return _sqrtv(px * px + py * py + pz * pz) - 1.0

        @plsc.parallel_loop(0, NSAMP)
        def _(j):
            setrow(sdf0, j, sdf_at(row(ubin, j)))

        setrow(cdf, 0, zf)
        oray = gi * (L * NOUT) + lane * NOUT

        srcs = [(ubin, sdf0), (spb_b, sdf_b), (spb_a, sdf_a), (spb_b, sdf_b)]
        dsts = [(spb_b, sdf_b), (spb_a, sdf_a), (spb_b, sdf_b), (None, None)]
        for it in range(NSTEP):
            S = NSAMP + NPER * it
            spb_src, sdf_src = srcs[it]
            spb_dst, sdf_dst = dsts[it]
            last = it == NSTEP - 1
            inv_s = BASEVAR * (2.0 ** it)

            def passA(s, carry):
                trans, pcraw, c = carry
                sdf_s = row(sdf_src, s)
                spb_s = row(spb_src, s)
                sdf_n = row(sdf_src, s + 1)
                spb_n = row(spb_src, s + 1)
                dlt = (spb_n - spb_s) * fmn
                cos = (sdf_n - sdf_s) / (dlt + 1e-5)
                cv = jnp.clip(jnp.minimum(pcraw, cos), -1e3, 0.0)
                mid = (sdf_s + sdf_n) * 0.5
                h = cv * dlt * 0.5
                ea = jnp.exp(jnp.minimum((h - mid) * inv_s, 40.0))
                eb = jnp.exp(jnp.minimum((-h - mid) * inv_s, 40.0))
                q = 1e-5 * ((1.0 + ea) * (1.0 + eb))
                alpha = (eb - ea + q) / (1.0 + eb + q)
                c = c + alpha * trans + 1e-5
                trans = trans * ((1.0 + 1e-7) - alpha)
                setrow(cdf, s + 1, c)
                return (trans, cos, c)

            carry0 = (jnp.ones((L,), jnp.float32), zf, zf)
            _, _, c_end = plsc.parallel_loop(0, S - 1, carry=carry0)(passA)
            ws = c_end + 1e-5
            setrow(cdf, S, ws)

            def sample_one(kf):
                u = jnp.broadcast_to(kf * (1.0 / 17.0) + (1.0 / 34.0), (L,)) * ws
                inds = _search_right(cdf, u, S + 1, lane)
                below = (inds - 1) * L + lane
                above = inds * L + lane
                g0 = plsc.load_gather(cdf, [below])
                g1 = plsc.load_gather(cdf, [above])
                b0 = plsc.load_gather(spb_src, [below])
                b1 = plsc.load_gather(spb_src, [above])
                t = jnp.clip((u - g0) / (g1 - g0), 0.0, 1.0)
                return b0 + t * (b1 - b0), inds

            @plsc.parallel_loop(0, NPER)
            def _(k):
                bv, pb = sample_one(lax.convert_element_type(k, jnp.float32))
                plsc.addupdate_scatter(delta, [pb * L + lane], onei)
                rb = pb + jnp.broadcast_to(k, (L,)).astype(jnp.int32)
                if last:
                    plsc.store_scatter(outc_v, [oray + rb], near + bv * fmn)
                else:
                    plsc.store_scatter(spb_dst, [rb * L + lane], bv)
                    plsc.store_scatter(sdf_dst, [rb * L + lane], sdf_at(bv))
            end_b, _ = sample_one(jnp.float32(NPER))

            def apass(i, cnt):
                dv = row(delta, i)
                setrow(delta, i, zi)
                cnt = cnt + dv
                r = cnt + jnp.broadcast_to(i, (L,)).astype(jnp.int32)
                av = row(spb_src, i)
                if last:
                    plsc.store_scatter(outc_v, [oray + r], near + av * fmn)
                else:
                    plsc.store_scatter(spb_dst, [r * L + lane], av)
                    plsc.store_scatter(sdf_dst, [r * L + lane], row(sdf_src, i))
                return cnt

            plsc.parallel_loop(0, S, carry=zi)(apass)
            setrow(delta, S, zi)

            endv = jnp.maximum(row(spb_src, S), end_b)
            if last:
                plsc.store_scatter(
                    outc_v, [oray + jnp.broadcast_to(SOUT, (L,)).astype(jnp.int32)],
                    near + endv * fmn)
            else:
                setrow(spb_dst, S + NPER, endv)

    def chunk_body(c, _):
        def group_body(gg, _):
            do_group(c * CH + gg, gg)
            return 0

        lax.fori_loop(0, CH, group_body, 0)
        pltpu.sync_copy(
            outc_v,
            out_hbm.at[pl.ds((wid * GPW + c * CH) * (L * NOUT), CH * L * NOUT)])
        return 0

    lax.fori_loop(0, NCHUNK, chunk_body, 0)


_mesh = plsc.VectorSubcoreMesh(core_axis_name="c", subcore_axis_name="s")

_sc_sampler = functools.partial(
    pl.kernel,
    out_type=jax.ShapeDtypeStruct((NRAYS * NOUT,), jnp.float32),
    mesh=_mesh,
    compiler_params=pltpu.CompilerParams(needs_layout_passes=False),
    scratch_types=[
        pltpu.VMEM((RPW * 3,), jnp.float32),
        pltpu.VMEM((RPW * 3,), jnp.float32),
        pltpu.VMEM((RPW,), jnp.float32),
        pltpu.VMEM((RPW,), jnp.float32),
        pltpu.VMEM((CH * L * NOUT,), jnp.float32),
        pltpu.VMEM((NOUT * L,), jnp.float32),
        pltpu.VMEM((SOUT * L,), jnp.float32),
        pltpu.VMEM((NOUT * L,), jnp.float32),
        pltpu.VMEM((SOUT * L,), jnp.float32),
        pltpu.VMEM((NOUT * L,), jnp.float32),
        pltpu.VMEM((NOUT * L,), jnp.int32),
        pltpu.VMEM(((NSAMP + 1) * L,), jnp.float32),
        pltpu.VMEM((NSAMP * L,), jnp.float32),
    ],
)(_sc_body)


def kernel(origins, directions, nears, fars):
    out = _sc_sampler(origins.reshape(-1), directions.reshape(-1),
                      nears.reshape(-1), fars.reshape(-1))
    return out.reshape(NRAYS, NOUT)


# speedup vs baseline: 1.1156x; 1.0136x over previous
import functools

import jax
import jax.numpy as jnp
from jax import lax
from jax.experimental import pallas as pl
from jax.experimental.pallas import tpu as pltpu
from jax.experimental.pallas import tpu_sc as plsc

NSAMP = 64
NSTEP = 4
NPER = 16
BASEVAR = 64.0
NRAYS = 65536
L = 16
NC, NS = 2, 16
NWORK = NC * NS
GROUPS = NRAYS // L
GPW = GROUPS // NWORK
RPW = GPW * L
CH = 16
NCHUNK = GPW // CH
SOUT = NSAMP + NSTEP * NPER
NOUT = SOUT + 1


def _sqrtv(a):
    i = lax.bitcast_convert_type(a, jnp.int32)
    x = lax.bitcast_convert_type(jnp.int32(0x5F3759DF) - (i >> 1), jnp.float32)
    for _ in range(3):
        x = x * (1.5 - 0.5 * a * x * x)
    return a * x


def _search_right(ref, v, length, lane):
    pos = jnp.zeros((L,), jnp.int32)
    step = 64
    while step >= 1:
        cand = jnp.minimum(pos + step, length - 1)
        av = plsc.load_gather(ref, [cand * L + lane])
        pos = jnp.where(av <= v, cand, pos)
        step //= 2
    return pos + 1


def _sc_body(org_hbm, dir_hbm, near_hbm, far_hbm, out_hbm,
             org_v, dir_v, near_v, far_v, outc_v,
             spb_a, sdf_a, spb_b, sdf_b, cdf, delta, ubin, sdf0, sem):
    cid = lax.axis_index("c")
    sid = lax.axis_index("s")
    wid = sid * NC + cid
    lane = lax.iota(jnp.int32, L)
    zf = jnp.zeros((L,), jnp.float32)
    zi = jnp.zeros((L,), jnp.int32)
    onei = jnp.ones((L,), jnp.int32)

    pltpu.sync_copy(org_hbm.at[pl.ds(wid * (RPW * 3), RPW * 3)], org_v)
    pltpu.sync_copy(dir_hbm.at[pl.ds(wid * (RPW * 3), RPW * 3)], dir_v)
    pltpu.sync_copy(near_hbm.at[pl.ds(wid * RPW, RPW)], near_v)
    pltpu.sync_copy(far_hbm.at[pl.ds(wid * RPW, RPW)], far_v)

    def row(ref, i):
        return ref[pl.ds(i * L, L)]

    def setrow(ref, i, v):
        ref[pl.ds(i * L, L)] = v

    @plsc.parallel_loop(0, NOUT)
    def _(i):
        setrow(delta, i, zi)

    @plsc.parallel_loop(0, NSAMP + 1)
    def _(j):
        setrow(ubin, j, jnp.broadcast_to(
            lax.convert_element_type(j, jnp.float32) * (1.0 / NSAMP), (L,)))

    def do_group(g, gi, obase):
        r3 = (g * L + lane) * 3
        ox = plsc.load_gather(org_v, [r3])
        oy = plsc.load_gather(org_v, [r3 + 1])
        oz = plsc.load_gather(org_v, [r3 + 2])
        dx = plsc.load_gather(dir_v, [r3])
        dy = plsc.load_gather(dir_v, [r3 + 1])
        dz = plsc.load_gather(dir_v, [r3 + 2])
        near = row(near_v, g)
        far = row(far_v, g)
        fmn = far - near

        def sdf_at(x):
            t = near + x * fmn
            px = ox + dx * t
            py = oy + dy * t
            pz = oz + dz * t
            return _sqrtv(px * px + py * py + pz * pz) - 1.0

        @plsc.parallel_loop(0, NSAMP)
        def _(j):
            setrow(sdf0, j, sdf_at(row(ubin, j)))

        setrow(cdf, 0, zf)
        oray = obase + gi * (L * NOUT) + lane * NOUT

        srcs = [(ubin, sdf0), (spb_b, sdf_b), (spb_a, sdf_a), (spb_b, sdf_b)]
        dsts = [(spb_b, sdf_b), (spb_a, sdf_a), (spb_b, sdf_b), (None, None)]
        for it in range(NSTEP):
            S = NSAMP + NPER * it
            spb_src, sdf_src = srcs[it]
            spb_dst, sdf_dst = dsts[it]
            last = it == NSTEP - 1
            inv_s = BASEVAR * (2.0 ** it)

            def passA(s, carry):
                trans, pcraw, c = carry
                sdf_s = row(sdf_src, s)
                spb_s = row(spb_src, s)
                sdf_n = row(sdf_src, s + 1)
                spb_n = row(spb_src, s + 1)
                dlt = (spb_n - spb_s) * fmn
                cos = (sdf_n - sdf_s) / (dlt + 1e-5)
                cv = jnp.clip(jnp.minimum(pcraw, cos), -1e3, 0.0)
                mid = (sdf_s + sdf_n) * 0.5
                h = cv * dlt * 0.5
                ea = jnp.exp(jnp.minimum((h - mid) * inv_s, 40.0))
                eb = jnp.exp(jnp.minimum((-h - mid) * inv_s, 40.0))
                q = 1e-5 * ((1.0 + ea) * (1.0 + eb))
                alpha = (eb - ea + q) / (1.0 + eb + q)
                c = c + alpha * trans + 1e-5
                trans = trans * ((1.0 + 1e-7) - alpha)
                setrow(cdf, s + 1, c)
                return (trans, cos, c)

            carry0 = (jnp.ones((L,), jnp.float32), zf, zf)
            _, _, c_end = plsc.parallel_loop(0, S - 1, carry=carry0)(passA)
            ws = c_end + 1e-5
            setrow(cdf, S, ws)

            def sample_one(kf):
                u = jnp.broadcast_to(kf * (1.0 / 17.0) + (1.0 / 34.0), (L,)) * ws
                inds = _search_right(cdf, u, S + 1, lane)
                below = (inds - 1) * L + lane
                above = inds * L + lane
                g0 = plsc.load_gather(cdf, [below])
                g1 = plsc.load_gather(cdf, [above])
                b0 = plsc.load_gather(spb_src, [below])
                b1 = plsc.load_gather(spb_src, [above])
                t = jnp.clip((u - g0) / (g1 - g0), 0.0, 1.0)
                return b0 + t * (b1 - b0), inds

            @plsc.parallel_loop(0, NPER)
            def _(k):
                bv, pb = sample_one(lax.convert_element_type(k, jnp.float32))
                plsc.addupdate_scatter(delta, [pb * L + lane], onei)
                rb = pb + jnp.broadcast_to(k, (L,)).astype(jnp.int32)
                if last:
                    plsc.store_scatter(outc_v, [oray + rb], near + bv * fmn)
                else:
                    plsc.store_scatter(spb_dst, [rb * L + lane], bv)
                    plsc.store_scatter(sdf_dst, [rb * L + lane], sdf_at(bv))
            end_b, _ = sample_one(jnp.float32(NPER))

            def apass(i, cnt):
                dv = row(delta, i)
                setrow(delta, i, zi)
                cnt = cnt + dv
                r = cnt + jnp.broadcast_to(i, (L,)).astype(jnp.int32)
                av = row(spb_src, i)
                if last:
                    plsc.store_scatter(outc_v, [oray + r], near + av * fmn)
                else:
                    plsc.store_scatter(spb_dst, [r * L + lane], av)
                    plsc.store_scatter(sdf_dst, [r * L + lane], row(sdf_src, i))
                return cnt

            plsc.parallel_loop(0, S, carry=zi)(apass)
            setrow(delta, S, zi)

            endv = jnp.maximum(row(spb_src, S), end_b)
            if last:
                plsc.store_scatter(
                    outc_v, [oray + jnp.broadcast_to(SOUT, (L,)).astype(jnp.int32)],
                    near + endv * fmn)
            else:
                setrow(spb_dst, S + NPER, endv)

    CHSZ = CH * L * NOUT

    def chunk_body(c, _):
        obase = lax.rem(c, 2) * CHSZ

        @pl.when(c >= 2)
        def _():
            pltpu.make_async_copy(out_hbm.at[pl.ds(0, CHSZ)],
                                  outc_v.at[pl.ds(0, CHSZ)], sem).wait()

        def group_body(gg, _):
            do_group(c * CH + gg, gg, obase)
            return 0

        lax.fori_loop(0, CH, group_body, 0)
        pltpu.async_copy(
            outc_v.at[pl.ds(obase, CHSZ)],
            out_hbm.at[pl.ds((wid * GPW + c * CH) * (L * NOUT), CHSZ)], sem)
        return 0

    lax.fori_loop(0, NCHUNK, chunk_body, 0)
    for _ in range(2):
        pltpu.make_async_copy(out_hbm.at[pl.ds(0, CHSZ)],
                              outc_v.at[pl.ds(0, CHSZ)], sem).wait()


_mesh = plsc.VectorSubcoreMesh(core_axis_name="c", subcore_axis_name="s")

_sc_sampler = functools.partial(
    pl.kernel,
    out_type=jax.ShapeDtypeStruct((NRAYS * NOUT,), jnp.float32),
    mesh=_mesh,
    compiler_params=pltpu.CompilerParams(needs_layout_passes=False),
    scratch_types=[
        pltpu.VMEM((RPW * 3,), jnp.float32),
        pltpu.VMEM((RPW * 3,), jnp.float32),
        pltpu.VMEM((RPW,), jnp.float32),
        pltpu.VMEM((RPW,), jnp.float32),
        pltpu.VMEM((2 * CH * L * NOUT,), jnp.float32),
        pltpu.VMEM((NOUT * L,), jnp.float32),
        pltpu.VMEM((SOUT * L,), jnp.float32),
        pltpu.VMEM((NOUT * L,), jnp.float32),
        pltpu.VMEM((SOUT * L,), jnp.float32),
        pltpu.VMEM((NOUT * L,), jnp.float32),
        pltpu.VMEM((NOUT * L,), jnp.int32),
        pltpu.VMEM(((NSAMP + 1) * L,), jnp.float32),
        pltpu.VMEM((NSAMP * L,), jnp.float32),
        pltpu.SemaphoreType.DMA,
    ],
)(_sc_body)


def kernel(origins, directions, nears, fars):
    out = _sc_sampler(origins.reshape(-1), directions.reshape(-1),
                      nears.reshape(-1), fars.reshape(-1))
    return out.reshape(NRAYS, NOUT)
